# speedup vs baseline: 1.8685x; 1.8685x over previous
import functools
import math

import jax
import jax.numpy as jnp
from jax import lax
from jax.experimental import pallas as pl
from jax.experimental.pallas import tpu as pltpu

_SMOOTHING = 0.1
_PAD = 1


def _body(tgt_ref, lp_ref, out_ref, *, eps, conf, c):
    pid = pl.program_id(0)
    blk = lp_ref[...]
    t = tgt_ref[...]
    rowsum = jnp.sum(blk, axis=1, keepdims=True)
    vb = blk[:, _PAD:_PAD + 1]
    cols = lax.broadcasted_iota(jnp.int32, blk.shape, 1)
    vt = jnp.sum(jnp.where(cols == t, blk, 0.0), axis=1, keepdims=True)
    contrib = jnp.where(
        t != _PAD, c - eps * rowsum + eps * vb + (eps - conf) * vt, 0.0
    )
    s = jnp.sum(contrib)

    @pl.when(pid == 0)
    def _():
        out_ref[0, 0] = 0.0

    out_ref[0, 0] += s


def kernel(log_probs, targets):
    lp = log_probs.reshape(-1, log_probs.shape[-1])
    n, v = lp.shape
    tgt = targets.reshape(-1, 1).astype(jnp.int32)
    rb = 64
    eps = _SMOOTHING / (v - 2)
    conf = 1.0 - _SMOOTHING
    c = (v - 2) * eps * math.log(eps) + conf * math.log(conf)
    out = pl.pallas_call(
        functools.partial(_body, eps=eps, conf=conf, c=c),
        grid=(n // rb,),
        in_specs=[
            pl.BlockSpec((rb, 1), lambda i: (i, 0)),
            pl.BlockSpec((rb, v), lambda i: (i, 0)),
        ],
        out_specs=pl.BlockSpec(
            (1, 1), lambda i: (0, 0), memory_space=pltpu.SMEM
        ),
        out_shape=jax.ShapeDtypeStruct((1, 1), jnp.float32),
    )(tgt, lp)
    return out[0, 0]
